# baseline (speedup 1.0000x reference)
import functools

import jax
import jax.numpy as jnp
from jax import lax
from jax.experimental import pallas as pl
from jax.experimental.pallas import tpu as pltpu
from jax.experimental.pallas import tpu_sc as plsc

D = 64
CHUNK = 128


@functools.lru_cache(maxsize=None)
def _make_gather(B):
    info = plsc.get_sparse_core_info()
    NC, NS = info.num_cores, info.num_subcores
    NW = NC * NS
    b_per_w = B // NW
    assert b_per_w * NW == B and b_per_w % CHUNK == 0
    n_chunks = b_per_w // CHUNK
    mesh = plsc.VectorSubcoreMesh(core_axis_name="c", subcore_axis_name="s")

    @functools.partial(
        pl.kernel,
        out_type=jax.ShapeDtypeStruct((NW, n_chunks, CHUNK, D), jnp.float32),
        mesh=mesh,
        scratch_types=[
            pltpu.VMEM((n_chunks, CHUNK), jnp.int32),
            pltpu.VMEM((CHUNK, D), jnp.float32),
            pltpu.SemaphoreType.DMA,
        ],
        compiler_params=pltpu.CompilerParams(use_tc_tiling_on_sc=False),
    )
    def k(table_hbm, idx_hbm, out_hbm, idx_v, rows_v, sem):
        wid = lax.axis_index("s") * NC + lax.axis_index("c")
        pltpu.sync_copy(idx_hbm.at[wid], idx_v)

        def step(j, carry):
            pltpu.async_copy(table_hbm.at[idx_v.at[j]], rows_v, sem).wait()
            pltpu.sync_copy(rows_v, out_hbm.at[wid, j])
            return carry

        lax.fori_loop(0, n_chunks, step, 0)

    return k, NW, n_chunks


def kernel(x, latent_tdirs):
    n, t = x.shape[0], x.shape[1]
    B = n * t
    k, NW, n_chunks = _make_gather(B)
    idx = x.reshape(NW, n_chunks, CHUNK)
    out = k(latent_tdirs, idx)
    return out.reshape(n, t, D)


# speedup vs baseline: 1.0490x; 1.0490x over previous
import functools

import jax
import jax.numpy as jnp
from jax import lax
from jax.experimental import pallas as pl
from jax.experimental.pallas import tpu as pltpu
from jax.experimental.pallas import tpu_sc as plsc

D = 64
CHUNK = 128
NBUF = 5


@functools.lru_cache(maxsize=None)
def _make_gather(B):
    info = plsc.get_sparse_core_info()
    NC, NS = info.num_cores, info.num_subcores
    NW = NC * NS
    b_per_w = B // NW
    assert b_per_w * NW == B and b_per_w % CHUNK == 0
    n_chunks = b_per_w // CHUNK
    assert n_chunks > NBUF
    mesh = plsc.VectorSubcoreMesh(core_axis_name="c", subcore_axis_name="s")

    @functools.partial(
        pl.kernel,
        out_type=jax.ShapeDtypeStruct((NW, n_chunks, CHUNK, D), jnp.float32),
        mesh=mesh,
        scratch_types=[
            pltpu.VMEM((n_chunks, CHUNK), jnp.int32),
            pltpu.VMEM((NBUF, CHUNK, D), jnp.float32),
            pltpu.SemaphoreType.DMA,
            pltpu.SemaphoreType.DMA,
        ],
        compiler_params=pltpu.CompilerParams(use_tc_tiling_on_sc=False),
    )
    def k(table_hbm, idx_hbm, out_hbm, idx_v, rows_v, gsem, osem):
        wid = lax.axis_index("s") * NC + lax.axis_index("c")
        pltpu.sync_copy(idx_hbm.at[wid], idx_v)

        def gstart(j, b):
            pltpu.async_copy(table_hbm.at[idx_v.at[j]], rows_v.at[b], gsem)

        def gwait(b):
            pltpu.make_async_copy(
                table_hbm.at[pl.ds(0, CHUNK)], rows_v.at[b], gsem).wait()

        def ostart(j, b):
            pltpu.async_copy(rows_v.at[b], out_hbm.at[wid, j], osem)

        def owait(b):
            pltpu.make_async_copy(
                rows_v.at[b], out_hbm.at[wid, 0], osem).wait()

        for b in range(NBUF):
            gstart(b, b)

        def step(j, carry):
            b = j % NBUF
            gwait(b)
            ostart(j, b)
            owait(b)
            gstart(j + NBUF, b)
            return carry

        lax.fori_loop(0, n_chunks - NBUF, step, 0)

        for j in range(n_chunks - NBUF, n_chunks):
            b = j % NBUF
            gwait(b)
            ostart(j, b)
        for j in range(n_chunks - NBUF, n_chunks):
            owait(j % NBUF)

    return k, NW, n_chunks


def kernel(x, latent_tdirs):
    n, t = x.shape[0], x.shape[1]
    B = n * t
    k, NW, n_chunks = _make_gather(B)
    idx = x.reshape(NW, n_chunks, CHUNK)
    out = k(latent_tdirs, idx)
    return out.reshape(n, t, D)


# speedup vs baseline: 1.0503x; 1.0013x over previous
import functools

import jax
import jax.numpy as jnp
from jax import lax
from jax.experimental import pallas as pl
from jax.experimental.pallas import tpu as pltpu
from jax.experimental.pallas import tpu_sc as plsc

D = 64
CHUNK = 256
NBUF = 4


@functools.lru_cache(maxsize=None)
def _make_gather(B):
    info = plsc.get_sparse_core_info()
    NC, NS = info.num_cores, info.num_subcores
    NW = NC * NS
    b_per_w = B // NW
    assert b_per_w * NW == B and b_per_w % CHUNK == 0
    n_chunks = b_per_w // CHUNK
    assert n_chunks > NBUF
    mesh = plsc.VectorSubcoreMesh(core_axis_name="c", subcore_axis_name="s")

    @functools.partial(
        pl.kernel,
        out_type=jax.ShapeDtypeStruct((NW, n_chunks, CHUNK, D), jnp.float32),
        mesh=mesh,
        scratch_types=[
            pltpu.VMEM((n_chunks, CHUNK), jnp.int32),
            pltpu.VMEM((NBUF, CHUNK, D), jnp.float32),
            pltpu.SemaphoreType.DMA,
            pltpu.SemaphoreType.DMA,
        ],
        compiler_params=pltpu.CompilerParams(use_tc_tiling_on_sc=False),
    )
    def k(table_hbm, idx_hbm, out_hbm, idx_v, rows_v, gsem, osem):
        wid = lax.axis_index("s") * NC + lax.axis_index("c")
        pltpu.sync_copy(idx_hbm.at[wid], idx_v)

        def gstart(j, b):
            pltpu.async_copy(table_hbm.at[idx_v.at[j]], rows_v.at[b], gsem)

        def gwait(b):
            pltpu.make_async_copy(
                table_hbm.at[pl.ds(0, CHUNK)], rows_v.at[b], gsem).wait()

        def ostart(j, b):
            pltpu.async_copy(rows_v.at[b], out_hbm.at[wid, j], osem)

        def owait(b):
            pltpu.make_async_copy(
                rows_v.at[b], out_hbm.at[wid, 0], osem).wait()

        for b in range(NBUF):
            gstart(b, b)

        def step(j, carry):
            b = j % NBUF
            gwait(b)
            ostart(j, b)
            owait(b)
            gstart(j + NBUF, b)
            return carry

        lax.fori_loop(0, n_chunks - NBUF, step, 0)

        for j in range(n_chunks - NBUF, n_chunks):
            b = j % NBUF
            gwait(b)
            ostart(j, b)
        for j in range(n_chunks - NBUF, n_chunks):
            owait(j % NBUF)

    return k, NW, n_chunks


def kernel(x, latent_tdirs):
    n, t = x.shape[0], x.shape[1]
    B = n * t
    k, NW, n_chunks = _make_gather(B)
    idx = x.reshape(NW, n_chunks, CHUNK)
    out = k(latent_tdirs, idx)
    return out.reshape(n, t, D)
